# Initial kernel scaffold; baseline (speedup 1.0000x reference)
import jax
import jax.numpy as jnp
from jax.experimental import pallas as pl


def kernel(tokens, emb, W, b):
    raise NotImplementedError("write your pallas kernel here")


# speedup vs baseline: 2.6547x; 2.6547x over previous
import functools

import jax
import jax.numpy as jnp
from jax import lax
from jax.experimental import pallas as pl
from jax.experimental.pallas import tpu as pltpu
from jax.experimental.pallas import tpu_sc as plsc

_NC = 2
_NS = 16
_NW = _NC * _NS

_B = 4 * 2048
_D = 1280
_BPW = _B // _NW
_CHUNK = 32
_NCHUNK = _BPW // _CHUNK

_VPAD = 40


def _table_body(emb_ref, w_ref, b_ref, out_ref):
    w_sum = w_ref[:, :512] + w_ref[:, 512:]
    acc = jax.lax.dot_general(
        emb_ref[:], w_sum,
        dimension_numbers=(((1,), (1,)), ((), ())),
        preferred_element_type=jnp.float32,
    )
    out_ref[:] = acc + b_ref[:]


def _compute_table(emb, w, b):
    emb_pad = jnp.zeros((_VPAD, 512), jnp.float32).at[:33].set(emb)
    return pl.pallas_call(
        _table_body,
        out_shape=jax.ShapeDtypeStruct((_VPAD, _D), jnp.float32),
    )(emb_pad, w, b.reshape(1, _D))


def _gather_body(tok_hbm, table_hbm, out_hbm, idx_v, buf0, buf1, sem0, sem1):
    wid = lax.axis_index("s") * _NC + lax.axis_index("c")
    base = wid * _BPW
    pltpu.sync_copy(tok_hbm.at[wid], idx_v)

    bufs = (buf0, buf1)
    sems = (sem0, sem1)
    copies = [None, None]
    copies[0] = pltpu.async_copy(table_hbm.at[idx_v.at[0]], bufs[0], sems[0])
    for c in range(_NCHUNK):
        s = c % 2
        if c + 1 < _NCHUNK:
            n = (c + 1) % 2
            copies[n] = pltpu.async_copy(
                table_hbm.at[idx_v.at[c + 1]], bufs[n], sems[n])
        copies[s].wait()
        pltpu.sync_copy(bufs[s], out_hbm.at[pl.ds(base + c * _CHUNK, _CHUNK)])


_gather = functools.partial(
    pl.kernel,
    out_type=jax.ShapeDtypeStruct((_B, _D), jnp.float32),
    mesh=plsc.VectorSubcoreMesh(
        core_axis_name="c", subcore_axis_name="s",
        num_cores=_NC, num_subcores=_NS),
    scratch_types=[
        pltpu.VMEM((_NCHUNK, _CHUNK), jnp.int32),
        pltpu.VMEM((_CHUNK, _D), jnp.float32),
        pltpu.VMEM((_CHUNK, _D), jnp.float32),
        pltpu.SemaphoreType.DMA,
        pltpu.SemaphoreType.DMA,
    ],
)(_gather_body)


def kernel(tokens, emb, W, b):
    table = _compute_table(emb, W, b)
    tok = tokens.astype(jnp.int32).reshape(_NW, _NCHUNK, _CHUNK)
    out = _gather(tok, table)
    return out.reshape(tokens.shape[0], tokens.shape[1], _D)
